# baseline (speedup 1.0000x reference)
import functools

import jax
import jax.numpy as jnp
from jax import lax
from jax.experimental import pallas as pl
from jax.experimental.pallas import tpu as pltpu
from jax.experimental.pallas import tpu_sc as plsc

_MARGIN = 0.01
_BIG = 1e30

_B, _N, _K = 16, 2048, 256
_NC, _NS, _L = 2, 16, 16
_NW = _NC * _NS
_HALF = _K // _NC
_NV = _N // _L
_KV = _HALF // _L
_UNR = 4
_SP_FILL = _UNR + 1
_T_FILL = 2
_SP_PAD = _N + (_SP_FILL + 1) * _L
_T_PAD = _HALF + (_T_FILL + 1) * _L


def _worker_body(scores_hbm, maskf_hbm, idx_hbm,
                 out_loss_hbm, out_pairs_hbm,
                 s_v, m_v, sp_v, idx_v, t_v, ol_v, op_v,
                 sem_s, sem_m, sem_i):
    c = lax.axis_index("c")
    s = lax.axis_index("s")
    wid = s * _NC + c
    row = s
    half = c

    cps = []
    for ch in range(_N // 128):
        cps.append(pltpu.async_copy(
            scores_hbm.at[row, pl.ds(ch * 128, 128)],
            s_v.at[pl.ds(ch * 128, 128)], sem_s))
        cps.append(pltpu.async_copy(
            maskf_hbm.at[row, pl.ds(ch * 128, 128)],
            m_v.at[pl.ds(ch * 128, 128)], sem_m))
    for ch in range(_K // 128):
        cps.append(pltpu.async_copy(
            idx_hbm.at[row, pl.ds(ch * 128, 128)],
            idx_v.at[pl.ds(ch * 128, 128)], sem_i))
    for cp in cps:
        cp.wait()

    zeros = jnp.zeros((_L,), jnp.float32)
    bigs = jnp.full((_L,), _BIG, jnp.float32)
    nbigs = jnp.full((_L,), -_BIG, jnp.float32)
    full = bigs > 0.0

    def tbody(j, ecnt):
        eidx = idx_v[pl.ds(half * _HALF + j * _L, _L)]
        es = plsc.load_gather(s_v, [eidx])
        ew = plsc.load_gather(m_v, [eidx])
        valid = ew > 0.0
        plsc.store_compressed(t_v.at[pl.ds(ecnt, _L)], es + _MARGIN, mask=valid)
        return ecnt + plsc.all_reduce_population_count(valid)[0]

    ecnt = lax.fori_loop(0, _KV, tbody, jnp.int32(0))
    for k in range(_T_FILL):
        plsc.store_compressed(t_v.at[pl.ds(ecnt + k * _L, _L)], nbigs, mask=full)

    def kbody0(j, _):
        iv = idx_v[pl.ds(j * _L, _L)]
        plsc.store_scatter(m_v, [iv], zeros)
        return 0

    lax.fori_loop(0, _K // _L, kbody0, 0)

    def sbody(i, scnt):
        for h in range(2):
            sl = s_v[pl.ds(i * 2 * _L + h * _L, _L)]
            ml = m_v[pl.ds(i * 2 * _L + h * _L, _L)]
            surv = ml > 0.0
            plsc.store_compressed(sp_v.at[pl.ds(scnt, _L)], sl, mask=surv)
            scnt = scnt + plsc.all_reduce_population_count(surv)[0]
        return scnt

    scnt = lax.fori_loop(0, _NV // 2, sbody, jnp.int32(0))
    for k in range(_SP_FILL):
        plsc.store_compressed(sp_v.at[pl.ds(scnt + k * _L, _L)], bigs, mask=full)

    kv = (ecnt + _L - 1) // _L
    nv = (scnt + _UNR * _L - 1) // (_UNR * _L)

    def kbody(r, accs):
        tvec = t_v[pl.ds(r * _L, _L)]
        ts = [tvec[l] for l in range(_L)]

        def ibody(i, iaccs):
            iaccs = list(iaccs)
            for q in range(_UNR):
                sp = sp_v[pl.ds(i * (_UNR * _L) + q * _L, _L)]
                for l in range(_L):
                    a = (q * _L + l) % _UNR
                    iaccs[a] = iaccs[a] + jnp.maximum(ts[l] - sp, 0.0)
            return tuple(iaccs)

        return lax.fori_loop(0, nv, ibody, accs)

    accs = lax.fori_loop(0, kv, kbody, (zeros,) * _UNR)
    acc = accs[0]
    for a in accs[1:]:
        acc = acc + a

    loss = jnp.sum(acc)
    pairs = ecnt.astype(jnp.float32) * scnt.astype(jnp.float32)
    ol_v[...] = jnp.full((_L,), 1.0 / _L, jnp.float32) * loss
    op_v[...] = jnp.full((_L,), 1.0 / _L, jnp.float32) * pairs
    pltpu.sync_copy(ol_v, out_loss_hbm.at[pl.ds(wid * _L, _L)])
    pltpu.sync_copy(op_v, out_pairs_hbm.at[pl.ds(wid * _L, _L)])


_sc_call = functools.partial(
    pl.kernel,
    out_type=[jax.ShapeDtypeStruct((_NW * _L,), jnp.float32),
              jax.ShapeDtypeStruct((_NW * _L,), jnp.float32)],
    mesh=plsc.VectorSubcoreMesh(core_axis_name="c", subcore_axis_name="s"),
    compiler_params=pltpu.CompilerParams(needs_layout_passes=False),
    scratch_types=[
        pltpu.VMEM((_N,), jnp.float32),
        pltpu.VMEM((_N,), jnp.float32),
        pltpu.VMEM((_SP_PAD,), jnp.float32),
        pltpu.VMEM((_K,), jnp.int32),
        pltpu.VMEM((_T_PAD,), jnp.float32),
        pltpu.VMEM((_L,), jnp.float32),
        pltpu.VMEM((_L,), jnp.float32),
        pltpu.SemaphoreType.DMA,
        pltpu.SemaphoreType.DMA,
        pltpu.SemaphoreType.DMA,
    ],
)(_worker_body)


def kernel(total_scores, eliminated_idx_list, mask):
    maskf = mask.astype(jnp.float32)
    out_loss, out_pairs = _sc_call(total_scores, maskf, eliminated_idx_list)
    total_loss = out_loss.sum()
    total_pairs = out_pairs.sum()
    return jnp.where(total_pairs > 0, total_loss / total_pairs, total_loss)


# speedup vs baseline: 1.0001x; 1.0001x over previous
import functools

import jax
import jax.numpy as jnp
from jax import lax
from jax.experimental import pallas as pl
from jax.experimental.pallas import tpu as pltpu
from jax.experimental.pallas import tpu_sc as plsc

_MARGIN = 0.01
_BIG = 1e30

_B, _N, _K = 16, 2048, 256
_NC, _NS, _L = 2, 16, 16
_NW = _NC * _NS
_HALF = _K // _NC
_NV = _N // _L
_KV = _HALF // _L
_UNR = 4
_NSEG = 4
_PIVOTS = (-0.6745, 0.0, 0.6745)
_SP_SEG = _N + 6 * _L
_T_SEG = _HALF + 3 * _L


def _seg_masks(x, base):
    lo1 = x < _PIVOTS[0]
    lo2 = x < _PIVOTS[1]
    lo3 = x < _PIVOTS[2]
    return (base & lo1,
            base & (~lo1) & lo2,
            base & (~lo2) & lo3,
            base & (~lo3))


def _worker_body(scores_hbm, maskf_hbm, idx_hbm,
                 out_loss_hbm, out_pairs_hbm,
                 s_v, m_v, sp_v, idx_v, t_v, ol_v, op_v,
                 sem_s, sem_m, sem_i):
    c = lax.axis_index("c")
    s = lax.axis_index("s")
    wid = s * _NC + c
    row = s
    half = c

    cps = []
    for ch in range(_N // 128):
        cps.append(pltpu.async_copy(
            scores_hbm.at[row, pl.ds(ch * 128, 128)],
            s_v.at[pl.ds(ch * 128, 128)], sem_s))
        cps.append(pltpu.async_copy(
            maskf_hbm.at[row, pl.ds(ch * 128, 128)],
            m_v.at[pl.ds(ch * 128, 128)], sem_m))
    for ch in range(_K // 128):
        cps.append(pltpu.async_copy(
            idx_hbm.at[row, pl.ds(ch * 128, 128)],
            idx_v.at[pl.ds(ch * 128, 128)], sem_i))
    for cp in cps:
        cp.wait()

    zeros = jnp.zeros((_L,), jnp.float32)
    bigs = jnp.full((_L,), _BIG, jnp.float32)
    nbigs = jnp.full((_L,), -_BIG, jnp.float32)
    full = bigs > 0.0
    i0 = jnp.int32(0)

    def tbody(j, carry):
        offs, sums = carry
        eidx = idx_v[pl.ds(half * _HALF + j * _L, _L)]
        es = plsc.load_gather(s_v, [eidx])
        ew = plsc.load_gather(m_v, [eidx])
        t = es + _MARGIN
        offs = list(offs)
        sums = list(sums)
        for g, mg in enumerate(_seg_masks(t, ew > 0.0)):
            plsc.store_compressed(
                t_v.at[pl.ds(g * _T_SEG + offs[g], _L)], t, mask=mg)
            offs[g] = offs[g] + plsc.all_reduce_population_count(mg)[0]
            sums[g] = sums[g] + jnp.where(mg, t, 0.0)
        return tuple(offs), tuple(sums)

    (ecnt, tsumv) = lax.fori_loop(
        0, _KV, tbody, ((i0,) * _NSEG, (zeros,) * _NSEG))
    tsum = [jnp.sum(v) for v in tsumv]
    for g in range(_NSEG):
        for k in range(2):
            plsc.store_compressed(
                t_v.at[pl.ds(g * _T_SEG + ecnt[g] + k * _L, _L)],
                nbigs, mask=full)

    def kbody0(j, _):
        iv = idx_v[pl.ds(j * _L, _L)]
        plsc.store_scatter(m_v, [iv], zeros)
        return 0

    lax.fori_loop(0, _K // _L, kbody0, 0)

    def sbody(i, carry):
        offs, sums = carry
        sl = s_v[pl.ds(i * _L, _L)]
        ml = m_v[pl.ds(i * _L, _L)]
        offs = list(offs)
        sums = list(sums)
        for g, mg in enumerate(_seg_masks(sl, ml > 0.0)):
            plsc.store_compressed(
                sp_v.at[pl.ds(g * _SP_SEG + offs[g], _L)], sl, mask=mg)
            offs[g] = offs[g] + plsc.all_reduce_population_count(mg)[0]
            sums[g] = sums[g] + jnp.where(mg, sl, 0.0)
        return tuple(offs), tuple(sums)

    (scnt, ssumv) = lax.fori_loop(
        0, _NV, sbody, ((i0,) * _NSEG, (zeros,) * _NSEG))
    ssum = [jnp.sum(v) for v in ssumv]
    for g in range(_NSEG):
        for k in range(_UNR + 1):
            plsc.store_compressed(
                sp_v.at[pl.ds(g * _SP_SEG + scnt[g] + k * _L, _L)],
                bigs, mask=full)

    accs = (zeros,) * _UNR
    bulk = jnp.float32(0.0)
    run_cnt = jnp.float32(0.0)
    run_sum = jnp.float32(0.0)
    for g in range(_NSEG):
        ecnt_f = ecnt[g].astype(jnp.float32)
        bulk = bulk + run_cnt * tsum[g] - ecnt_f * run_sum
        run_cnt = run_cnt + scnt[g].astype(jnp.float32)
        run_sum = run_sum + ssum[g]

        kv = (ecnt[g] + _L - 1) // _L
        nv = (scnt[g] + _UNR * _L - 1) // (_UNR * _L)
        tbase = g * _T_SEG
        spbase = g * _SP_SEG

        def kbody(r, kaccs, tbase=tbase, spbase=spbase, nv=nv):
            tvec = t_v[pl.ds(tbase + r * _L, _L)]
            ts = [tvec[l] for l in range(_L)]

            def ibody(i, iaccs):
                iaccs = list(iaccs)
                for q in range(_UNR):
                    sp = sp_v[pl.ds(spbase + i * (_UNR * _L) + q * _L, _L)]
                    for l in range(_L):
                        a = (q * _L + l) % _UNR
                        iaccs[a] = iaccs[a] + jnp.maximum(ts[l] - sp, 0.0)
                return tuple(iaccs)

            return lax.fori_loop(0, nv, ibody, kaccs)

        accs = lax.fori_loop(0, kv, kbody, accs)

    acc = accs[0]
    for a in accs[1:]:
        acc = acc + a

    loss = jnp.sum(acc) + bulk
    tot_e = ecnt[0] + ecnt[1] + ecnt[2] + ecnt[3]
    tot_s = scnt[0] + scnt[1] + scnt[2] + scnt[3]
    pairs = tot_e.astype(jnp.float32) * tot_s.astype(jnp.float32)
    ol_v[...] = jnp.full((_L,), 1.0 / _L, jnp.float32) * loss
    op_v[...] = jnp.full((_L,), 1.0 / _L, jnp.float32) * pairs
    pltpu.sync_copy(ol_v, out_loss_hbm.at[pl.ds(wid * _L, _L)])
    pltpu.sync_copy(op_v, out_pairs_hbm.at[pl.ds(wid * _L, _L)])


_sc_call = functools.partial(
    pl.kernel,
    out_type=[jax.ShapeDtypeStruct((_NW * _L,), jnp.float32),
              jax.ShapeDtypeStruct((_NW * _L,), jnp.float32)],
    mesh=plsc.VectorSubcoreMesh(core_axis_name="c", subcore_axis_name="s"),
    compiler_params=pltpu.CompilerParams(needs_layout_passes=False),
    scratch_types=[
        pltpu.VMEM((_N,), jnp.float32),
        pltpu.VMEM((_N,), jnp.float32),
        pltpu.VMEM((_NSEG * _SP_SEG,), jnp.float32),
        pltpu.VMEM((_K,), jnp.int32),
        pltpu.VMEM((_NSEG * _T_SEG,), jnp.float32),
        pltpu.VMEM((_L,), jnp.float32),
        pltpu.VMEM((_L,), jnp.float32),
        pltpu.SemaphoreType.DMA,
        pltpu.SemaphoreType.DMA,
        pltpu.SemaphoreType.DMA,
    ],
)(_worker_body)


def kernel(total_scores, eliminated_idx_list, mask):
    maskf = mask.astype(jnp.float32)
    out_loss, out_pairs = _sc_call(total_scores, maskf, eliminated_idx_list)
    total_loss = out_loss.sum()
    total_pairs = out_pairs.sum()
    return jnp.where(total_pairs > 0, total_loss / total_pairs, total_loss)


# speedup vs baseline: 1.0495x; 1.0494x over previous
import functools

import jax
import jax.numpy as jnp
from jax import lax
from jax.experimental import pallas as pl
from jax.experimental.pallas import tpu as pltpu
from jax.experimental.pallas import tpu_sc as plsc

_MARGIN = 0.01
_BIG = 1e30

_B, _N, _K = 16, 2048, 256
_NC, _NS, _L = 2, 16, 16
_NW = _NC * _NS
_HALF = _K // _NC
_NV = _N // _L
_KV = _HALF // _L
_UNR = 4
_NSEG = 4
_PIVOTS = (-0.6745, 0.0, 0.6745)
_SP_SEG = _N + 6 * _L
_T_SEG = _HALF + 3 * _L


def _seg_masks(x, base):
    lo1 = x < _PIVOTS[0]
    lo2 = x < _PIVOTS[1]
    lo3 = x < _PIVOTS[2]
    return (base & lo1,
            base & (~lo1) & lo2,
            base & (~lo2) & lo3,
            base & (~lo3))


def _worker_body(scores_hbm, maskf_hbm, idx_hbm,
                 out_loss_hbm, out_pairs_hbm,
                 s_v, m_v, sp_v, idx_v, t_v, ol_v, op_v,
                 cnt_sm, sum_sm,
                 sem_s, sem_m, sem_i):
    c = lax.axis_index("c")
    s = lax.axis_index("s")
    wid = s * _NC + c
    row = s
    half = c

    def _score_cp(ch):
        return pltpu.make_async_copy(
            scores_hbm.at[row, pl.ds(ch * 128, 128)],
            s_v.at[pl.ds(ch * 128, 128)], sem_s)

    def _mask_cp(ch):
        return pltpu.make_async_copy(
            maskf_hbm.at[row, pl.ds(ch * 128, 128)],
            m_v.at[pl.ds(ch * 128, 128)], sem_m)

    def _idx_cp(ch):
        return pltpu.make_async_copy(
            idx_hbm.at[row, pl.ds(ch * 128, 128)],
            idx_v.at[pl.ds(ch * 128, 128)], sem_i)

    def dstart(ch, _):
        _score_cp(ch).start()
        _mask_cp(ch).start()
        return 0

    lax.fori_loop(0, _N // 128, dstart, 0)
    for ch in range(_K // 128):
        _idx_cp(ch).start()
    for ch in range(_K // 128):
        _idx_cp(ch).wait()

    def dwait(ch, _):
        _score_cp(ch).wait()
        _mask_cp(ch).wait()
        return 0

    lax.fori_loop(0, _N // 128, dwait, 0)

    zeros = jnp.zeros((_L,), jnp.float32)
    bigs = jnp.full((_L,), _BIG, jnp.float32)
    nbigs = jnp.full((_L,), -_BIG, jnp.float32)
    full = bigs > 0.0
    i0 = jnp.int32(0)

    def tbody(j, carry):
        offs, sums = carry
        eidx = idx_v[pl.ds(half * _HALF + j * _L, _L)]
        es = plsc.load_gather(s_v, [eidx])
        ew = plsc.load_gather(m_v, [eidx])
        t = es + _MARGIN
        offs = list(offs)
        sums = list(sums)
        for g, mg in enumerate(_seg_masks(t, ew > 0.0)):
            plsc.store_compressed(
                t_v.at[pl.ds(g * _T_SEG + offs[g], _L)], t, mask=mg)
            offs[g] = offs[g] + plsc.all_reduce_population_count(mg)[0]
            sums[g] = sums[g] + jnp.where(mg, t, 0.0)
        return tuple(offs), tuple(sums)

    (ecnt, tsumv) = lax.fori_loop(
        0, _KV, tbody, ((i0,) * _NSEG, (zeros,) * _NSEG))

    def kbody0(j, _):
        iv = idx_v[pl.ds(j * _L, _L)]
        plsc.store_scatter(m_v, [iv], zeros)
        return 0

    lax.fori_loop(0, _K // _L, kbody0, 0)

    def sbody(i, carry):
        offs, sums = carry
        sl = s_v[pl.ds(i * _L, _L)]
        ml = m_v[pl.ds(i * _L, _L)]
        offs = list(offs)
        sums = list(sums)
        for g, mg in enumerate(_seg_masks(sl, ml > 0.0)):
            plsc.store_compressed(
                sp_v.at[pl.ds(g * _SP_SEG + offs[g], _L)], sl, mask=mg)
            offs[g] = offs[g] + plsc.all_reduce_population_count(mg)[0]
            sums[g] = sums[g] + jnp.where(mg, sl, 0.0)
        return tuple(offs), tuple(sums)

    (scnt, ssumv) = lax.fori_loop(
        0, _NV, sbody, ((i0,) * _NSEG, (zeros,) * _NSEG))

    for g in range(_NSEG):
        cnt_sm[g] = ecnt[g]
        cnt_sm[_NSEG + g] = scnt[g]
        sum_sm[g] = jnp.sum(tsumv[g])
        sum_sm[_NSEG + g] = jnp.sum(ssumv[g])

    def fillb(g, _):
        ec = cnt_sm[g]
        for k in range(2):
            plsc.store_compressed(
                t_v.at[pl.ds(g * _T_SEG + ec + k * _L, _L)], nbigs, mask=full)
        sc = cnt_sm[_NSEG + g]
        for k in range(_UNR + 1):
            plsc.store_compressed(
                sp_v.at[pl.ds(g * _SP_SEG + sc + k * _L, _L)], bigs, mask=full)
        return 0

    lax.fori_loop(0, _NSEG, fillb, 0)

    def segb(g, carry):
        accs, bulk, run_cnt, run_sum = carry
        ec = cnt_sm[g]
        sc = cnt_sm[_NSEG + g]
        tsum_g = sum_sm[g]
        ssum_g = sum_sm[_NSEG + g]
        bulk = bulk + run_cnt * tsum_g - ec.astype(jnp.float32) * run_sum
        run_cnt = run_cnt + sc.astype(jnp.float32)
        run_sum = run_sum + ssum_g

        kv = (ec + _L - 1) // _L
        nv = (sc + _UNR * _L - 1) // (_UNR * _L)
        tbase = g * _T_SEG
        spbase = g * _SP_SEG

        def kbody(r, kaccs):
            tvec = t_v[pl.ds(tbase + r * _L, _L)]
            ts = [tvec[l] for l in range(_L)]

            def ibody(i, iaccs):
                iaccs = list(iaccs)
                for q in range(_UNR):
                    sp = sp_v[pl.ds(spbase + i * (_UNR * _L) + q * _L, _L)]
                    for l in range(_L):
                        a = (q * _L + l) % _UNR
                        iaccs[a] = iaccs[a] + jnp.maximum(ts[l] - sp, 0.0)
                return tuple(iaccs)

            return lax.fori_loop(0, nv, ibody, kaccs)

        accs = lax.fori_loop(0, kv, kbody, accs)
        return accs, bulk, run_cnt, run_sum

    accs, bulk, run_cnt, run_sum = lax.fori_loop(
        0, _NSEG, segb,
        ((zeros,) * _UNR, jnp.float32(0.0), jnp.float32(0.0), jnp.float32(0.0)))

    acc = accs[0]
    for a in accs[1:]:
        acc = acc + a

    loss = jnp.sum(acc) + bulk
    tot_e = ecnt[0] + ecnt[1] + ecnt[2] + ecnt[3]
    pairs = tot_e.astype(jnp.float32) * run_cnt
    ol_v[...] = jnp.full((_L,), 1.0 / _L, jnp.float32) * loss
    op_v[...] = jnp.full((_L,), 1.0 / _L, jnp.float32) * pairs
    pltpu.sync_copy(ol_v, out_loss_hbm.at[pl.ds(wid * _L, _L)])
    pltpu.sync_copy(op_v, out_pairs_hbm.at[pl.ds(wid * _L, _L)])


_sc_call = functools.partial(
    pl.kernel,
    out_type=[jax.ShapeDtypeStruct((_NW * _L,), jnp.float32),
              jax.ShapeDtypeStruct((_NW * _L,), jnp.float32)],
    mesh=plsc.VectorSubcoreMesh(core_axis_name="c", subcore_axis_name="s"),
    compiler_params=pltpu.CompilerParams(needs_layout_passes=False),
    scratch_types=[
        pltpu.VMEM((_N,), jnp.float32),
        pltpu.VMEM((_N,), jnp.float32),
        pltpu.VMEM((_NSEG * _SP_SEG,), jnp.float32),
        pltpu.VMEM((_K,), jnp.int32),
        pltpu.VMEM((_NSEG * _T_SEG,), jnp.float32),
        pltpu.VMEM((_L,), jnp.float32),
        pltpu.VMEM((_L,), jnp.float32),
        pltpu.SMEM((2 * _NSEG,), jnp.int32),
        pltpu.SMEM((2 * _NSEG,), jnp.float32),
        pltpu.SemaphoreType.DMA,
        pltpu.SemaphoreType.DMA,
        pltpu.SemaphoreType.DMA,
    ],
)(_worker_body)


def kernel(total_scores, eliminated_idx_list, mask):
    maskf = mask.astype(jnp.float32)
    out_loss, out_pairs = _sc_call(total_scores, maskf, eliminated_idx_list)
    total_loss = out_loss.sum()
    total_pairs = out_pairs.sum()
    return jnp.where(total_pairs > 0, total_loss / total_pairs, total_loss)


# speedup vs baseline: 1.0570x; 1.0072x over previous
import functools

import jax
import jax.numpy as jnp
from jax import lax
from jax.experimental import pallas as pl
from jax.experimental.pallas import tpu as pltpu
from jax.experimental.pallas import tpu_sc as plsc

_MARGIN = 0.01
_BIG = 1e30

_B, _N, _K = 16, 2048, 256
_NC, _NS, _L = 2, 16, 16
_NW = _NC * _NS
_HALF = _K // _NC
_NV = _N // _L
_KV = _HALF // _L
_UNR = 2
_NSEG = 4
_PIVOTS = (-0.6745, 0.0, 0.6745)
_SP_SEG = _N + 6 * _L
_T_SEG = _HALF + 3 * _L


def _seg_masks(x, base):
    lo1 = x < _PIVOTS[0]
    lo2 = x < _PIVOTS[1]
    lo3 = x < _PIVOTS[2]
    return (base & lo1,
            base & (~lo1) & lo2,
            base & (~lo2) & lo3,
            base & (~lo3))


def _worker_body(scores_hbm, maskf_hbm, idx_hbm,
                 out_loss_hbm, out_pairs_hbm,
                 s_v, m_v, sp_v, idx_v, t_v, ol_v, op_v,
                 cnt_sm, sum_sm,
                 sem_s, sem_m, sem_i):
    c = lax.axis_index("c")
    s = lax.axis_index("s")
    wid = s * _NC + c
    row = s
    half = c

    def _score_cp(ch):
        return pltpu.make_async_copy(
            scores_hbm.at[row, pl.ds(ch * 128, 128)],
            s_v.at[pl.ds(ch * 128, 128)], sem_s)

    def _mask_cp(ch):
        return pltpu.make_async_copy(
            maskf_hbm.at[row, pl.ds(ch * 128, 128)],
            m_v.at[pl.ds(ch * 128, 128)], sem_m)

    def _idx_cp(ch):
        return pltpu.make_async_copy(
            idx_hbm.at[row, pl.ds(ch * 128, 128)],
            idx_v.at[pl.ds(ch * 128, 128)], sem_i)

    def dstart(ch, _):
        _score_cp(ch).start()
        _mask_cp(ch).start()
        return 0

    lax.fori_loop(0, _N // 128, dstart, 0)
    for ch in range(_K // 128):
        _idx_cp(ch).start()
    for ch in range(_K // 128):
        _idx_cp(ch).wait()

    def dwait(ch, _):
        _score_cp(ch).wait()
        _mask_cp(ch).wait()
        return 0

    lax.fori_loop(0, _N // 128, dwait, 0)

    zeros = jnp.zeros((_L,), jnp.float32)
    bigs = jnp.full((_L,), _BIG, jnp.float32)
    nbigs = jnp.full((_L,), -_BIG, jnp.float32)
    full = bigs > 0.0
    i0 = jnp.int32(0)

    def tbody(j, carry):
        offs, sums = carry
        eidx = idx_v[pl.ds(half * _HALF + j * _L, _L)]
        es = plsc.load_gather(s_v, [eidx])
        ew = plsc.load_gather(m_v, [eidx])
        t = es + _MARGIN
        offs = list(offs)
        sums = list(sums)
        for g, mg in enumerate(_seg_masks(t, ew > 0.0)):
            plsc.store_compressed(
                t_v.at[pl.ds(g * _T_SEG + offs[g], _L)], t, mask=mg)
            offs[g] = offs[g] + plsc.all_reduce_population_count(mg)[0]
            sums[g] = sums[g] + jnp.where(mg, t, 0.0)
        return tuple(offs), tuple(sums)

    (ecnt, tsumv) = lax.fori_loop(
        0, _KV, tbody, ((i0,) * _NSEG, (zeros,) * _NSEG))

    def kbody0(j, _):
        iv = idx_v[pl.ds(j * _L, _L)]
        plsc.store_scatter(m_v, [iv], zeros)
        return 0

    lax.fori_loop(0, _K // _L, kbody0, 0)

    def sbody(i, carry):
        offs, sums = carry
        sl = s_v[pl.ds(i * _L, _L)]
        ml = m_v[pl.ds(i * _L, _L)]
        offs = list(offs)
        sums = list(sums)
        for g, mg in enumerate(_seg_masks(sl, ml > 0.0)):
            plsc.store_compressed(
                sp_v.at[pl.ds(g * _SP_SEG + offs[g], _L)], sl, mask=mg)
            offs[g] = offs[g] + plsc.all_reduce_population_count(mg)[0]
            sums[g] = sums[g] + jnp.where(mg, sl, 0.0)
        return tuple(offs), tuple(sums)

    (scnt, ssumv) = lax.fori_loop(
        0, _NV, sbody, ((i0,) * _NSEG, (zeros,) * _NSEG))

    for g in range(_NSEG):
        cnt_sm[g] = ecnt[g]
        cnt_sm[_NSEG + g] = scnt[g]
        sum_sm[g] = jnp.sum(tsumv[g])
        sum_sm[_NSEG + g] = jnp.sum(ssumv[g])

    def fillb(g, _):
        ec = cnt_sm[g]
        for k in range(2):
            plsc.store_compressed(
                t_v.at[pl.ds(g * _T_SEG + ec + k * _L, _L)], nbigs, mask=full)
        sc = cnt_sm[_NSEG + g]
        for k in range(_UNR + 1):
            plsc.store_compressed(
                sp_v.at[pl.ds(g * _SP_SEG + sc + k * _L, _L)], bigs, mask=full)
        return 0

    lax.fori_loop(0, _NSEG, fillb, 0)

    def segb(g, carry):
        accs, bulk, run_cnt, run_sum = carry
        ec = cnt_sm[g]
        sc = cnt_sm[_NSEG + g]
        tsum_g = sum_sm[g]
        ssum_g = sum_sm[_NSEG + g]
        bulk = bulk + run_cnt * tsum_g - ec.astype(jnp.float32) * run_sum
        run_cnt = run_cnt + sc.astype(jnp.float32)
        run_sum = run_sum + ssum_g

        kv = (ec + _L - 1) // _L
        nv = (sc + _UNR * _L - 1) // (_UNR * _L)
        tbase = g * _T_SEG
        spbase = g * _SP_SEG

        def kbody(r, kaccs):
            tvec = t_v[pl.ds(tbase + r * _L, _L)]
            ts = [tvec[l] for l in range(_L)]

            def ibody(i, iaccs):
                iaccs = list(iaccs)
                for q in range(_UNR):
                    sp = sp_v[pl.ds(spbase + i * (_UNR * _L) + q * _L, _L)]
                    for l in range(_L):
                        a = (q * _L + l) % _UNR
                        iaccs[a] = iaccs[a] + jnp.maximum(ts[l] - sp, 0.0)
                return tuple(iaccs)

            return lax.fori_loop(0, nv, ibody, kaccs)

        accs = lax.fori_loop(0, kv, kbody, accs)
        return accs, bulk, run_cnt, run_sum

    accs, bulk, run_cnt, run_sum = lax.fori_loop(
        0, _NSEG, segb,
        ((zeros,) * _UNR, jnp.float32(0.0), jnp.float32(0.0), jnp.float32(0.0)))

    acc = accs[0]
    for a in accs[1:]:
        acc = acc + a

    loss = jnp.sum(acc) + bulk
    tot_e = ecnt[0] + ecnt[1] + ecnt[2] + ecnt[3]
    pairs = tot_e.astype(jnp.float32) * run_cnt
    ol_v[...] = jnp.full((_L,), 1.0 / _L, jnp.float32) * loss
    op_v[...] = jnp.full((_L,), 1.0 / _L, jnp.float32) * pairs
    pltpu.sync_copy(ol_v, out_loss_hbm.at[pl.ds(wid * _L, _L)])
    pltpu.sync_copy(op_v, out_pairs_hbm.at[pl.ds(wid * _L, _L)])


_sc_call = functools.partial(
    pl.kernel,
    out_type=[jax.ShapeDtypeStruct((_NW * _L,), jnp.float32),
              jax.ShapeDtypeStruct((_NW * _L,), jnp.float32)],
    mesh=plsc.VectorSubcoreMesh(core_axis_name="c", subcore_axis_name="s"),
    compiler_params=pltpu.CompilerParams(needs_layout_passes=False),
    scratch_types=[
        pltpu.VMEM((_N,), jnp.float32),
        pltpu.VMEM((_N,), jnp.float32),
        pltpu.VMEM((_NSEG * _SP_SEG,), jnp.float32),
        pltpu.VMEM((_K,), jnp.int32),
        pltpu.VMEM((_NSEG * _T_SEG,), jnp.float32),
        pltpu.VMEM((_L,), jnp.float32),
        pltpu.VMEM((_L,), jnp.float32),
        pltpu.SMEM((2 * _NSEG,), jnp.int32),
        pltpu.SMEM((2 * _NSEG,), jnp.float32),
        pltpu.SemaphoreType.DMA,
        pltpu.SemaphoreType.DMA,
        pltpu.SemaphoreType.DMA,
    ],
)(_worker_body)


def kernel(total_scores, eliminated_idx_list, mask):
    maskf = mask.astype(jnp.float32)
    out_loss, out_pairs = _sc_call(total_scores, maskf, eliminated_idx_list)
    total_loss = out_loss.sum()
    total_pairs = out_pairs.sum()
    return jnp.where(total_pairs > 0, total_loss / total_pairs, total_loss)


# speedup vs baseline: 1.0628x; 1.0055x over previous
import functools

import jax
import jax.numpy as jnp
from jax import lax
from jax.experimental import pallas as pl
from jax.experimental.pallas import tpu as pltpu
from jax.experimental.pallas import tpu_sc as plsc

_MARGIN = 0.01
_BIG = 1e30

_B, _N, _K = 16, 2048, 256
_NC, _NS, _L = 2, 16, 16
_NW = _NC * _NS
_HALF = _K // _NC
_NV = _N // _L
_KV = _HALF // _L
_UNR = 2
_NSEG = 4
_PIVOTS = (-0.6745, 0.0, 0.6745)
_SP_SEG = _N + 6 * _L
_T_SEG = _HALF + 3 * _L


def _seg_masks(x, base):
    lo1 = x < _PIVOTS[0]
    lo2 = x < _PIVOTS[1]
    lo3 = x < _PIVOTS[2]
    return (base & lo1,
            base & (~lo1) & lo2,
            base & (~lo2) & lo3,
            base & (~lo3))


def _worker_body(scores_hbm, maskf_hbm, idx_hbm,
                 out_loss_hbm, out_pairs_hbm,
                 s_v, m_v, sp_v, idx_v, t_v, ol_v, op_v,
                 cnt_sm, sum_sm,
                 sem_s, sem_m, sem_i):
    c = lax.axis_index("c")
    s = lax.axis_index("s")
    wid = s * _NC + c
    row = s
    half = c

    def _score_cp(ch):
        return pltpu.make_async_copy(
            scores_hbm.at[row, pl.ds(ch * 128, 128)],
            s_v.at[pl.ds(ch * 128, 128)], sem_s)

    def _mask_cp(ch):
        return pltpu.make_async_copy(
            maskf_hbm.at[row, pl.ds(ch * 128, 128)],
            m_v.at[pl.ds(ch * 128, 128)], sem_m)

    def _idx_cp(ch):
        return pltpu.make_async_copy(
            idx_hbm.at[row, pl.ds(ch * 128, 128)],
            idx_v.at[pl.ds(ch * 128, 128)], sem_i)

    def dstart(ch, _):
        _score_cp(ch).start()
        _mask_cp(ch).start()
        return 0

    lax.fori_loop(0, _N // 128, dstart, 0)
    for ch in range(_K // 128):
        _idx_cp(ch).start()
    for ch in range(_K // 128):
        _idx_cp(ch).wait()

    pltpu.make_async_copy(scores_hbm.at[row], s_v, sem_s).wait()
    pltpu.make_async_copy(maskf_hbm.at[row], m_v, sem_m).wait()

    zeros = jnp.zeros((_L,), jnp.float32)
    bigs = jnp.full((_L,), _BIG, jnp.float32)
    nbigs = jnp.full((_L,), -_BIG, jnp.float32)
    full = bigs > 0.0
    i0 = jnp.int32(0)

    def tbody(j, carry):
        offs, sums = carry
        eidx = idx_v[pl.ds(half * _HALF + j * _L, _L)]
        es = plsc.load_gather(s_v, [eidx])
        ew = plsc.load_gather(m_v, [eidx])
        t = es + _MARGIN
        offs = list(offs)
        sums = list(sums)
        for g, mg in enumerate(_seg_masks(t, ew > 0.0)):
            plsc.store_compressed(
                t_v.at[pl.ds(g * _T_SEG + offs[g], _L)], t, mask=mg)
            offs[g] = offs[g] + plsc.all_reduce_population_count(mg)[0]
            sums[g] = sums[g] + jnp.where(mg, t, 0.0)
        return tuple(offs), tuple(sums)

    (ecnt, tsumv) = lax.fori_loop(
        0, _KV, tbody, ((i0,) * _NSEG, (zeros,) * _NSEG))

    def kbody0(j, _):
        iv = idx_v[pl.ds(j * _L, _L)]
        plsc.store_scatter(m_v, [iv], zeros)
        return 0

    lax.fori_loop(0, _K // _L, kbody0, 0)

    def sbody(i, carry):
        offs, sums = carry
        offs = list(offs)
        sums = list(sums)
        for h in range(2):
            sl = s_v[pl.ds(i * 2 * _L + h * _L, _L)]
            ml = m_v[pl.ds(i * 2 * _L + h * _L, _L)]
            for g, mg in enumerate(_seg_masks(sl, ml > 0.0)):
                plsc.store_compressed(
                    sp_v.at[pl.ds(g * _SP_SEG + offs[g], _L)], sl, mask=mg)
                offs[g] = offs[g] + plsc.all_reduce_population_count(mg)[0]
                sums[g] = sums[g] + jnp.where(mg, sl, 0.0)
        return tuple(offs), tuple(sums)

    (scnt, ssumv) = lax.fori_loop(
        0, _NV // 2, sbody, ((i0,) * _NSEG, (zeros,) * _NSEG))

    for g in range(_NSEG):
        cnt_sm[g] = ecnt[g]
        cnt_sm[_NSEG + g] = scnt[g]
        sum_sm[g] = jnp.sum(tsumv[g])
        sum_sm[_NSEG + g] = jnp.sum(ssumv[g])

    def fillb(g, _):
        ec = cnt_sm[g]
        for k in range(2):
            plsc.store_compressed(
                t_v.at[pl.ds(g * _T_SEG + ec + k * _L, _L)], nbigs, mask=full)
        sc = cnt_sm[_NSEG + g]
        for k in range(_UNR + 1):
            plsc.store_compressed(
                sp_v.at[pl.ds(g * _SP_SEG + sc + k * _L, _L)], bigs, mask=full)
        return 0

    lax.fori_loop(0, _NSEG, fillb, 0)

    def segb(g, carry):
        accs, bulk, run_cnt, run_sum = carry
        ec = cnt_sm[g]
        sc = cnt_sm[_NSEG + g]
        tsum_g = sum_sm[g]
        ssum_g = sum_sm[_NSEG + g]
        bulk = bulk + run_cnt * tsum_g - ec.astype(jnp.float32) * run_sum
        run_cnt = run_cnt + sc.astype(jnp.float32)
        run_sum = run_sum + ssum_g

        kv = (ec + _L - 1) // _L
        nv = (sc + _UNR * _L - 1) // (_UNR * _L)
        tbase = g * _T_SEG
        spbase = g * _SP_SEG

        def kbody(r, kaccs):
            tvec = t_v[pl.ds(tbase + r * _L, _L)]
            ts = [tvec[l] for l in range(_L)]

            def ibody(i, iaccs):
                iaccs = list(iaccs)
                for q in range(_UNR):
                    sp = sp_v[pl.ds(spbase + i * (_UNR * _L) + q * _L, _L)]
                    for l in range(_L):
                        a = (q * _L + l) % _UNR
                        iaccs[a] = iaccs[a] + jnp.maximum(ts[l] - sp, 0.0)
                return tuple(iaccs)

            return lax.fori_loop(0, nv, ibody, kaccs)

        accs = lax.fori_loop(0, kv, kbody, accs)
        return accs, bulk, run_cnt, run_sum

    accs, bulk, run_cnt, run_sum = lax.fori_loop(
        0, _NSEG, segb,
        ((zeros,) * _UNR, jnp.float32(0.0), jnp.float32(0.0), jnp.float32(0.0)))

    acc = accs[0]
    for a in accs[1:]:
        acc = acc + a

    loss = jnp.sum(acc) + bulk
    tot_e = ecnt[0] + ecnt[1] + ecnt[2] + ecnt[3]
    pairs = tot_e.astype(jnp.float32) * run_cnt
    ol_v[...] = jnp.full((_L,), 1.0 / _L, jnp.float32) * loss
    op_v[...] = jnp.full((_L,), 1.0 / _L, jnp.float32) * pairs
    pltpu.sync_copy(ol_v, out_loss_hbm.at[pl.ds(wid * _L, _L)])
    pltpu.sync_copy(op_v, out_pairs_hbm.at[pl.ds(wid * _L, _L)])


_sc_call = functools.partial(
    pl.kernel,
    out_type=[jax.ShapeDtypeStruct((_NW * _L,), jnp.float32),
              jax.ShapeDtypeStruct((_NW * _L,), jnp.float32)],
    mesh=plsc.VectorSubcoreMesh(core_axis_name="c", subcore_axis_name="s"),
    compiler_params=pltpu.CompilerParams(needs_layout_passes=False),
    scratch_types=[
        pltpu.VMEM((_N,), jnp.float32),
        pltpu.VMEM((_N,), jnp.float32),
        pltpu.VMEM((_NSEG * _SP_SEG,), jnp.float32),
        pltpu.VMEM((_K,), jnp.int32),
        pltpu.VMEM((_NSEG * _T_SEG,), jnp.float32),
        pltpu.VMEM((_L,), jnp.float32),
        pltpu.VMEM((_L,), jnp.float32),
        pltpu.SMEM((2 * _NSEG,), jnp.int32),
        pltpu.SMEM((2 * _NSEG,), jnp.float32),
        pltpu.SemaphoreType.DMA,
        pltpu.SemaphoreType.DMA,
        pltpu.SemaphoreType.DMA,
    ],
)(_worker_body)


def kernel(total_scores, eliminated_idx_list, mask):
    maskf = mask.astype(jnp.float32)
    out_loss, out_pairs = _sc_call(total_scores, maskf, eliminated_idx_list)
    total_loss = out_loss.sum()
    total_pairs = out_pairs.sum()
    return jnp.where(total_pairs > 0, total_loss / total_pairs, total_loss)


# speedup vs baseline: 1.0753x; 1.0117x over previous
import functools

import jax
import jax.numpy as jnp
from jax import lax
from jax.experimental import pallas as pl
from jax.experimental.pallas import tpu as pltpu
from jax.experimental.pallas import tpu_sc as plsc

_MARGIN = 0.01
_BIG = 1e30

_B, _N, _K = 16, 2048, 256
_NC, _NS, _L = 2, 16, 16
_NW = _NC * _NS
_HALF = _K // _NC
_NV = _N // _L
_KV = _HALF // _L
_UNR = 2
_NSEG = 4
_PIVOTS = (-0.6745, 0.0, 0.6745)
_SP_SEG = _N + 6 * _L
_T_SEG = _HALF + 3 * _L


def _seg_masks(x, base):
    lo1 = x < _PIVOTS[0]
    lo2 = x < _PIVOTS[1]
    lo3 = x < _PIVOTS[2]
    return (base & lo1,
            base & (~lo1) & lo2,
            base & (~lo2) & lo3,
            base & (~lo3))


def _worker_body(scores_hbm, maskf_hbm, idx_hbm,
                 out_hbm,
                 s_v, m_v, sp_v, idx_v, t_v, ol_v, op_v,
                 cnt_sm, sum_sm,
                 sem_s, sem_m, sem_i):
    c = lax.axis_index("c")
    s = lax.axis_index("s")
    wid = s * _NC + c
    row = s
    half = c

    def _score_cp(ch):
        return pltpu.make_async_copy(
            scores_hbm.at[row, pl.ds(ch * 128, 128)],
            s_v.at[pl.ds(ch * 128, 128)], sem_s)

    def _mask_cp(ch):
        return pltpu.make_async_copy(
            maskf_hbm.at[row, pl.ds(ch * 128, 128)],
            m_v.at[pl.ds(ch * 128, 128)], sem_m)

    def _idx_cp(ch):
        return pltpu.make_async_copy(
            idx_hbm.at[row, pl.ds(ch * 128, 128)],
            idx_v.at[pl.ds(ch * 128, 128)], sem_i)

    def dstart(ch, _):
        _score_cp(ch).start()
        _mask_cp(ch).start()
        return 0

    lax.fori_loop(0, _N // 128, dstart, 0)
    for ch in range(_K // 128):
        _idx_cp(ch).start()
    for ch in range(_K // 128):
        _idx_cp(ch).wait()

    pltpu.make_async_copy(scores_hbm.at[row], s_v, sem_s).wait()
    pltpu.make_async_copy(maskf_hbm.at[row], m_v, sem_m).wait()

    zeros = jnp.zeros((_L,), jnp.float32)
    bigs = jnp.full((_L,), _BIG, jnp.float32)
    nbigs = jnp.full((_L,), -_BIG, jnp.float32)
    full = bigs > 0.0
    i0 = jnp.int32(0)

    def tbody(j, carry):
        offs, sums = carry
        eidx = idx_v[pl.ds(half * _HALF + j * _L, _L)]
        es = plsc.load_gather(s_v, [eidx])
        ew = plsc.load_gather(m_v, [eidx])
        t = es + _MARGIN
        offs = list(offs)
        sums = list(sums)
        for g, mg in enumerate(_seg_masks(t, ew > 0.0)):
            plsc.store_compressed(
                t_v.at[pl.ds(g * _T_SEG + offs[g], _L)], t, mask=mg)
            offs[g] = offs[g] + plsc.all_reduce_population_count(mg)[0]
            sums[g] = sums[g] + jnp.where(mg, t, 0.0)
        return tuple(offs), tuple(sums)

    (ecnt, tsumv) = lax.fori_loop(
        0, _KV, tbody, ((i0,) * _NSEG, (zeros,) * _NSEG))

    def kbody0(j, _):
        iv = idx_v[pl.ds(j * _L, _L)]
        plsc.store_scatter(m_v, [iv], zeros)
        return 0

    lax.fori_loop(0, _K // _L, kbody0, 0)

    def sbody(i, carry):
        offs, sums = carry
        offs = list(offs)
        sums = list(sums)
        for h in range(2):
            sl = s_v[pl.ds(i * 2 * _L + h * _L, _L)]
            ml = m_v[pl.ds(i * 2 * _L + h * _L, _L)]
            for g, mg in enumerate(_seg_masks(sl, ml > 0.0)):
                plsc.store_compressed(
                    sp_v.at[pl.ds(g * _SP_SEG + offs[g], _L)], sl, mask=mg)
                offs[g] = offs[g] + plsc.all_reduce_population_count(mg)[0]
                sums[g] = sums[g] + jnp.where(mg, sl, 0.0)
        return tuple(offs), tuple(sums)

    (scnt, ssumv) = lax.fori_loop(
        0, _NV // 2, sbody, ((i0,) * _NSEG, (zeros,) * _NSEG))

    for g in range(_NSEG):
        cnt_sm[g] = ecnt[g]
        cnt_sm[_NSEG + g] = scnt[g]
        sum_sm[g] = jnp.sum(tsumv[g])
        sum_sm[_NSEG + g] = jnp.sum(ssumv[g])

    def fillb(g, _):
        ec = cnt_sm[g]
        for k in range(2):
            plsc.store_compressed(
                t_v.at[pl.ds(g * _T_SEG + ec + k * _L, _L)], nbigs, mask=full)
        sc = cnt_sm[_NSEG + g]
        for k in range(_UNR + 1):
            plsc.store_compressed(
                sp_v.at[pl.ds(g * _SP_SEG + sc + k * _L, _L)], bigs, mask=full)
        return 0

    lax.fori_loop(0, _NSEG, fillb, 0)

    def segb(g, carry):
        accs, bulk, run_cnt, run_sum = carry
        ec = cnt_sm[g]
        sc = cnt_sm[_NSEG + g]
        tsum_g = sum_sm[g]
        ssum_g = sum_sm[_NSEG + g]
        bulk = bulk + run_cnt * tsum_g - ec.astype(jnp.float32) * run_sum
        run_cnt = run_cnt + sc.astype(jnp.float32)
        run_sum = run_sum + ssum_g

        kv = (ec + _L - 1) // _L
        nv = (sc + _UNR * _L - 1) // (_UNR * _L)
        tbase = g * _T_SEG
        spbase = g * _SP_SEG

        def kbody(r, kaccs):
            tvec = t_v[pl.ds(tbase + r * _L, _L)]
            ts = [tvec[l] for l in range(_L)]

            def ibody(i, iaccs):
                iaccs = list(iaccs)
                for q in range(_UNR):
                    sp = sp_v[pl.ds(spbase + i * (_UNR * _L) + q * _L, _L)]
                    for l in range(_L):
                        a = (q * _L + l) % _UNR
                        iaccs[a] = iaccs[a] + jnp.maximum(ts[l] - sp, 0.0)
                return tuple(iaccs)

            return lax.fori_loop(0, nv, ibody, kaccs)

        accs = lax.fori_loop(0, kv, kbody, accs)
        return accs, bulk, run_cnt, run_sum

    accs, bulk, run_cnt, run_sum = lax.fori_loop(
        0, _NSEG, segb,
        ((zeros,) * _UNR, jnp.float32(0.0), jnp.float32(0.0), jnp.float32(0.0)))

    acc = accs[0]
    for a in accs[1:]:
        acc = acc + a

    loss = jnp.sum(acc) + bulk
    tot_e = ecnt[0] + ecnt[1] + ecnt[2] + ecnt[3]
    pairs = tot_e.astype(jnp.float32) * run_cnt
    ol_v[...] = jnp.full((_L,), 1.0 / _L, jnp.float32) * loss
    op_v[...] = jnp.full((_L,), 1.0 / _L, jnp.float32) * pairs
    pltpu.sync_copy(ol_v, out_hbm.at[pl.ds(wid * _L, _L)])
    pltpu.sync_copy(op_v, out_hbm.at[pl.ds(_NW * _L + wid * _L, _L)])


_sc_call = functools.partial(
    pl.kernel,
    out_type=jax.ShapeDtypeStruct((2 * _NW * _L,), jnp.float32),
    mesh=plsc.VectorSubcoreMesh(core_axis_name="c", subcore_axis_name="s"),
    compiler_params=pltpu.CompilerParams(needs_layout_passes=False),
    scratch_types=[
        pltpu.VMEM((_N,), jnp.float32),
        pltpu.VMEM((_N,), jnp.float32),
        pltpu.VMEM((_NSEG * _SP_SEG,), jnp.float32),
        pltpu.VMEM((_K,), jnp.int32),
        pltpu.VMEM((_NSEG * _T_SEG,), jnp.float32),
        pltpu.VMEM((_L,), jnp.float32),
        pltpu.VMEM((_L,), jnp.float32),
        pltpu.SMEM((2 * _NSEG,), jnp.int32),
        pltpu.SMEM((2 * _NSEG,), jnp.float32),
        pltpu.SemaphoreType.DMA,
        pltpu.SemaphoreType.DMA,
        pltpu.SemaphoreType.DMA,
    ],
)(_worker_body)


def kernel(total_scores, eliminated_idx_list, mask):
    maskf = mask.astype(jnp.float32)
    out = _sc_call(total_scores, maskf, eliminated_idx_list)
    total_loss = out[:_NW * _L].sum()
    total_pairs = out[_NW * _L:].sum()
    return jnp.where(total_pairs > 0, total_loss / total_pairs, total_loss)
